# baseline (speedup 1.0000x reference)
import functools

import jax
import jax.numpy as jnp
from jax import lax
from jax.experimental import pallas as pl
from jax.experimental.pallas import tpu as pltpu
from jax.experimental.pallas import tpu_sc as plsc

B = 16384
F = 26
VOCAB = 100000
D = 16

NC = 2
NS = 16
NW = NC * NS
BPW = B // NW

_mesh = plsc.VectorSubcoreMesh(core_axis_name="c", subcore_axis_name="s")


@functools.partial(
    pl.kernel,
    out_type=jax.ShapeDtypeStruct((B, D), jnp.float32),
    mesh=_mesh,
    scratch_types=[
        pltpu.VMEM((F, BPW), jnp.int32),
        pltpu.VMEM((BPW, D), jnp.float32),
        pltpu.SemaphoreType.DMA,
    ],
    compiler_params=pltpu.CompilerParams(use_tc_tiling_on_sc=False),
)
def _encode(xt_hbm, tables_hbm, out_hbm, idx_v, acc_v, sem):
    wid = lax.axis_index("s") * NC + lax.axis_index("c")
    base = wid * BPW
    tab = tables_hbm
    pltpu.sync_copy(xt_hbm.at[:, pl.ds(base, BPW)], idx_v)
    pltpu.async_copy(tab.at[idx_v.at[0]], acc_v, sem).wait()
    copies = [
        pltpu.async_copy(tab.at[idx_v.at[f]], acc_v, sem, add=True)
        for f in range(1, F)
    ]
    for c in copies:
        c.wait()
    pltpu.sync_copy(acc_v, out_hbm.at[pl.ds(base, BPW), :])


def kernel(x, tables):
    xt = (x + VOCAB * jnp.arange(F, dtype=jnp.int32)[None, :]).T
    return _encode(xt, tables.reshape(F * VOCAB, D))


# speedup vs baseline: 4.9112x; 4.9112x over previous
import functools

import jax
import jax.numpy as jnp
from jax import lax
from jax.experimental import pallas as pl
from jax.experimental.pallas import tpu as pltpu
from jax.experimental.pallas import tpu_sc as plsc

B = 16384
F = 26
VOCAB = 100000
D = 16

NC = 2
NS = 16
NW = NC * NS
NHALF = 2
FPW = F // NHALF
BC = 8192
LANES = 16

_mesh = plsc.VectorSubcoreMesh(core_axis_name="c", subcore_axis_name="s")


@functools.partial(
    pl.kernel,
    out_type=jax.ShapeDtypeStruct((NHALF, D, B), jnp.float32),
    mesh=_mesh,
    scratch_types=[
        pltpu.VMEM((VOCAB,), jnp.float32),
        pltpu.VMEM((B,), jnp.float32),
        pltpu.VMEM((BC,), jnp.int32),
        pltpu.SemaphoreType.DMA,
    ],
    compiler_params=pltpu.CompilerParams(needs_layout_passes=False),
)
def _encode(tsw_hbm, xt_hbm, out_hbm, tab_v, acc_v, idx_v, sem):
    w = lax.axis_index("s") * NC + lax.axis_index("c")
    d = w % D
    half = w // D
    for k, f in enumerate(range(0, F, NHALF)):
        fld = f + half
        pltpu.sync_copy(tsw_hbm.at[fld, d, :], tab_v)
        for cb in range(B // BC):
            pltpu.sync_copy(xt_hbm.at[pl.ds(fld * B + cb * BC, BC)], idx_v)

            @pl.loop(0, BC // LANES)
            def _gather(i, _k=k, _cb=cb):
                idx = idx_v[pl.ds(i * LANES, LANES)]
                g = plsc.load_gather(tab_v, [idx])
                sl = pl.ds(_cb * BC + i * LANES, LANES)
                if _k == 0:
                    acc_v[sl] = g
                else:
                    acc_v[sl] = acc_v[sl] + g

    pltpu.sync_copy(acc_v, out_hbm.at[half, d, :])


def kernel(x, tables):
    tsw = jnp.swapaxes(tables, 1, 2)
    xt = x.T.reshape(-1)
    parts = _encode(tsw, xt)
    return (parts[0] + parts[1]).T


# speedup vs baseline: 8.0395x; 1.6370x over previous
import functools

import jax
import jax.numpy as jnp
from jax import lax
from jax.experimental import pallas as pl
from jax.experimental.pallas import tpu as pltpu
from jax.experimental.pallas import tpu_sc as plsc

B = 16384
F = 26
VOCAB = 100000
D = 16

NC = 2
NS = 16
NW = NC * NS
NHALF = 2
FPW = F // NHALF
BC = 4096
LANES = 16

_mesh = plsc.VectorSubcoreMesh(core_axis_name="c", subcore_axis_name="s")


@functools.partial(
    pl.kernel,
    out_type=jax.ShapeDtypeStruct((NHALF, D, B), jnp.float32),
    mesh=_mesh,
    scratch_types=[
        pltpu.VMEM((VOCAB,), jnp.float32),
        pltpu.VMEM((B,), jnp.float32),
        pltpu.VMEM((2, BC), jnp.int32),
        pltpu.SemaphoreType.DMA,
        pltpu.SemaphoreType.DMA,
    ],
    compiler_params=pltpu.CompilerParams(needs_layout_passes=False),
)
def _encode(tsw_hbm, xt_hbm, out_hbm, tab_v, acc_v, idx_v, sem_t, sem_i):
    w = lax.axis_index("s") * NC + lax.axis_index("c")
    d = w % D
    half = w // D
    nbc = B // BC

    def start_idx_copy(fld, cb):
        return pltpu.async_copy(
            xt_hbm.at[pl.ds(fld * B + cb * BC, BC)], idx_v.at[cb % 2], sem_i
        )

    fields = [f + half for f in range(0, F, NHALF)]
    tab_copy = pltpu.async_copy(tsw_hbm.at[fields[0], d, :], tab_v, sem_t)
    idx_copy = start_idx_copy(fields[0], 0)
    tab_copy.wait()
    for k, fld in enumerate(fields):
        for cb in range(nbc):
            idx_copy.wait()
            if cb + 1 < nbc:
                idx_copy = start_idx_copy(fld, cb + 1)
            elif k + 1 < len(fields):
                idx_copy = start_idx_copy(fields[k + 1], 0)

            @plsc.parallel_loop(0, BC // LANES, unroll=8)
            def _gather(i, _k=k, _cb=cb):
                idx = idx_v[_cb % 2, pl.ds(i * LANES, LANES)]
                g = plsc.load_gather(tab_v, [idx])
                sl = pl.ds(_cb * BC + i * LANES, LANES)
                if _k == 0:
                    acc_v[sl] = g
                else:
                    plsc.addupdate(acc_v.at[sl], g)

        if k + 1 < len(fields):
            pltpu.async_copy(tsw_hbm.at[fields[k + 1], d, :], tab_v, sem_t).wait()

    pltpu.sync_copy(acc_v, out_hbm.at[half, d, :])


def kernel(x, tables):
    tsw = jnp.swapaxes(tables, 1, 2)
    xt = x.T.reshape(-1)
    parts = _encode(tsw, xt)
    return (parts[0] + parts[1]).T
